# Initial kernel scaffold; baseline (speedup 1.0000x reference)
import jax
import jax.numpy as jnp
from jax.experimental import pallas as pl


def kernel(x, rel_times):
    raise NotImplementedError("write your pallas kernel here")


# speedup vs baseline: 5.1847x; 5.1847x over previous
import functools

import jax
import jax.numpy as jnp
import numpy as np
from jax import lax
from jax.experimental import pallas as pl
from jax.experimental.pallas import tpu as pltpu
from jax.experimental.pallas import tpu_sc as plsc

D_MODEL = 128
MAX_LEN = 5000

NUM_CORES = 2
NUM_SUBCORES = 16
NUM_WORKERS = NUM_CORES * NUM_SUBCORES

CHUNK = 256


def _build_pe_np():
    position = np.arange(MAX_LEN, dtype=np.float32)[:, None]
    div_term = np.exp(
        np.arange(0, D_MODEL, 2, dtype=np.float32) * (-np.log(10000.0) / D_MODEL)
    ).astype(np.float32)
    pe_input = (position * div_term).astype(np.float32)
    pe = np.zeros((MAX_LEN, D_MODEL), dtype=np.float32)
    pe[:, ::2] = np.sin(pe_input)
    pe[:, 1::2] = np.cos(pe_input)
    return pe


_PE_TABLE = _build_pe_np()


def _make_sc_call(n_rows: int):
    rows_per_worker = n_rows // NUM_WORKERS
    n_chunks = rows_per_worker // CHUNK
    assert rows_per_worker * NUM_WORKERS == n_rows
    assert n_chunks * CHUNK == rows_per_worker

    mesh = plsc.VectorSubcoreMesh(
        core_axis_name="c", subcore_axis_name="s",
        num_cores=NUM_CORES, num_subcores=NUM_SUBCORES,
    )

    @functools.partial(
        pl.kernel,
        out_type=jax.ShapeDtypeStruct((n_rows, D_MODEL), jnp.float32),
        mesh=mesh,
        scratch_types=[
            pltpu.VMEM((CHUNK,), jnp.int32),
            pltpu.VMEM((CHUNK, D_MODEL), jnp.float32),
            pltpu.SemaphoreType.DMA,
        ],
    )
    def sc_call(x_hbm, pe_hbm, idx_hbm, out_hbm, idx_v, row_v, sem):
        wid = lax.axis_index("s") * NUM_CORES + lax.axis_index("c")
        w_base = wid * rows_per_worker

        def body(g, carry):
            base = w_base + g * CHUNK
            pltpu.sync_copy(idx_hbm.at[pl.ds(base, CHUNK)], idx_v)
            pltpu.sync_copy(x_hbm.at[pl.ds(base, CHUNK)], row_v)
            pltpu.async_copy(pe_hbm.at[idx_v], row_v, sem, add=True).wait()
            pltpu.sync_copy(row_v, out_hbm.at[pl.ds(base, CHUNK)])
            return carry

        lax.fori_loop(0, n_chunks, body, 0)

    return sc_call


def kernel(x, rel_times):
    b, t, d = x.shape
    n_rows = b * t
    x2 = x.reshape(n_rows, d)
    idx = rel_times.reshape(n_rows).astype(jnp.int32)
    pe = jnp.asarray(_PE_TABLE)
    out = _make_sc_call(n_rows)(x2, pe, idx)
    return out.reshape(b, t, d)


# speedup vs baseline: 6.0547x; 1.1678x over previous
import functools

import jax
import jax.numpy as jnp
import numpy as np
from jax import lax
from jax.experimental import pallas as pl
from jax.experimental.pallas import tpu as pltpu
from jax.experimental.pallas import tpu_sc as plsc

D_MODEL = 128
MAX_LEN = 5000

NUM_CORES = 2
NUM_SUBCORES = 16
NUM_WORKERS = NUM_CORES * NUM_SUBCORES

CHUNK = 128
NBUF = 4


def _build_pe_np():
    position = np.arange(MAX_LEN, dtype=np.float32)[:, None]
    div_term = np.exp(
        np.arange(0, D_MODEL, 2, dtype=np.float32) * (-np.log(10000.0) / D_MODEL)
    ).astype(np.float32)
    pe_input = (position * div_term).astype(np.float32)
    pe = np.zeros((MAX_LEN, D_MODEL), dtype=np.float32)
    pe[:, ::2] = np.sin(pe_input)
    pe[:, 1::2] = np.cos(pe_input)
    return pe


_PE_TABLE = _build_pe_np()


def _make_sc_call(n_rows: int):
    rows_per_worker = n_rows // NUM_WORKERS
    n_chunks = rows_per_worker // CHUNK
    n_outer = n_chunks // NBUF
    assert rows_per_worker * NUM_WORKERS == n_rows
    assert n_chunks * CHUNK == rows_per_worker
    assert n_outer * NBUF == n_chunks and n_outer >= 2

    mesh = plsc.VectorSubcoreMesh(
        core_axis_name="c", subcore_axis_name="s",
        num_cores=NUM_CORES, num_subcores=NUM_SUBCORES,
    )

    @functools.partial(
        pl.kernel,
        out_type=jax.ShapeDtypeStruct((n_rows, D_MODEL), jnp.float32),
        mesh=mesh,
        scratch_types=[
            pltpu.VMEM((rows_per_worker,), jnp.int32),
            [pltpu.VMEM((CHUNK, D_MODEL), jnp.float32) for _ in range(NBUF)],
            [pltpu.SemaphoreType.DMA for _ in range(NBUF)],
            [pltpu.SemaphoreType.DMA for _ in range(NBUF)],
            [pltpu.SemaphoreType.DMA for _ in range(NBUF)],
        ],
    )
    def sc_call(x_hbm, pe_hbm, idx_hbm, out_hbm, idx_all, rows, sem_x, sem_g, sem_w):
        wid = lax.axis_index("s") * NUM_CORES + lax.axis_index("c")
        w_base = wid * rows_per_worker

        pltpu.sync_copy(idx_hbm.at[pl.ds(w_base, rows_per_worker)], idx_all)

        def issue_load(local_chunk, b):
            pltpu.async_copy(
                x_hbm.at[pl.ds(w_base + local_chunk * CHUNK, CHUNK)],
                rows[b], sem_x[b])

        def process(local_chunk, b):
            pltpu.make_async_copy(
                x_hbm.at[pl.ds(w_base + local_chunk * CHUNK, CHUNK)],
                rows[b], sem_x[b]).wait()
            pltpu.async_copy(
                pe_hbm.at[idx_all.at[pl.ds(local_chunk * CHUNK, CHUNK)]],
                rows[b], sem_g[b], add=True).wait()
            pltpu.async_copy(
                rows[b], out_hbm.at[pl.ds(w_base + local_chunk * CHUNK, CHUNK)],
                sem_w[b])

        for b in range(NBUF):
            issue_load(b, b)

        def body(g, carry):
            for b in range(NBUF):
                local_chunk = g * NBUF + b
                process(local_chunk, b)
                pltpu.make_async_copy(
                    rows[b],
                    out_hbm.at[pl.ds(w_base + local_chunk * CHUNK, CHUNK)],
                    sem_w[b]).wait()
                issue_load(local_chunk + NBUF, b)
            return carry

        lax.fori_loop(0, n_outer - 1, body, 0)

        for b in range(NBUF):
            local_chunk = (n_outer - 1) * NBUF + b
            process(local_chunk, b)
        for b in range(NBUF):
            local_chunk = (n_outer - 1) * NBUF + b
            pltpu.make_async_copy(
                rows[b],
                out_hbm.at[pl.ds(w_base + local_chunk * CHUNK, CHUNK)],
                sem_w[b]).wait()

    return sc_call


def kernel(x, rel_times):
    b, t, d = x.shape
    n_rows = b * t
    x2 = x.reshape(n_rows, d)
    idx = rel_times.reshape(n_rows).astype(jnp.int32)
    pe = jnp.asarray(_PE_TABLE)
    out = _make_sc_call(n_rows)(x2, pe, idx)
    return out.reshape(b, t, d)


# speedup vs baseline: 6.9691x; 1.1510x over previous
import functools

import jax
import jax.numpy as jnp
import numpy as np
from jax import lax
from jax.experimental import pallas as pl
from jax.experimental.pallas import tpu as pltpu
from jax.experimental.pallas import tpu_sc as plsc

D_MODEL = 128
MAX_LEN = 5000

NUM_CORES = 2
NUM_SUBCORES = 16
NUM_WORKERS = NUM_CORES * NUM_SUBCORES

CHUNK = 128
NBUF = 4
LEAD = 2


def _build_pe_np():
    position = np.arange(MAX_LEN, dtype=np.float32)[:, None]
    div_term = np.exp(
        np.arange(0, D_MODEL, 2, dtype=np.float32) * (-np.log(10000.0) / D_MODEL)
    ).astype(np.float32)
    pe_input = (position * div_term).astype(np.float32)
    pe = np.zeros((MAX_LEN, D_MODEL), dtype=np.float32)
    pe[:, ::2] = np.sin(pe_input)
    pe[:, 1::2] = np.cos(pe_input)
    return pe


_PE_TABLE = _build_pe_np()


def _make_sc_call(n_rows: int):
    rows_per_worker = n_rows // NUM_WORKERS
    n_chunks = rows_per_worker // CHUNK
    n_steady = n_chunks - 2 * LEAD
    assert rows_per_worker * NUM_WORKERS == n_rows
    assert n_chunks * CHUNK == rows_per_worker
    assert n_steady % NBUF == 0 and n_steady >= NBUF

    mesh = plsc.VectorSubcoreMesh(
        core_axis_name="c", subcore_axis_name="s",
        num_cores=NUM_CORES, num_subcores=NUM_SUBCORES,
    )

    @functools.partial(
        pl.kernel,
        out_type=jax.ShapeDtypeStruct((n_rows, D_MODEL), jnp.float32),
        mesh=mesh,
        scratch_types=[
            pltpu.VMEM((rows_per_worker,), jnp.int32),
            [pltpu.VMEM((CHUNK, D_MODEL), jnp.float32) for _ in range(NBUF)],
            [pltpu.SemaphoreType.DMA for _ in range(NBUF)],
            [pltpu.SemaphoreType.DMA for _ in range(NBUF)],
            [pltpu.SemaphoreType.DMA for _ in range(NBUF)],
        ],
    )
    def sc_call(x_hbm, pe_hbm, idx_hbm, out_hbm, idx_all, rows, sem_x, sem_g, sem_w):
        wid = lax.axis_index("s") * NUM_CORES + lax.axis_index("c")
        w_base = wid * rows_per_worker

        pltpu.sync_copy(idx_hbm.at[pl.ds(w_base, rows_per_worker)], idx_all)

        def issue_load(c, b):
            pltpu.async_copy(
                x_hbm.at[pl.ds(w_base + c * CHUNK, CHUNK)], rows[b], sem_x[b])

        def wait_load(c, b):
            pltpu.make_async_copy(
                x_hbm.at[pl.ds(w_base + c * CHUNK, CHUNK)], rows[b],
                sem_x[b]).wait()

        def issue_store(c, b):
            pltpu.async_copy(
                rows[b], out_hbm.at[pl.ds(w_base + c * CHUNK, CHUNK)], sem_w[b])

        def wait_store(c, b):
            pltpu.make_async_copy(
                rows[b], out_hbm.at[pl.ds(w_base + c * CHUNK, CHUNK)],
                sem_w[b]).wait()

        def process(c, b):
            wait_load(c, b)
            pltpu.async_copy(
                pe_hbm.at[idx_all.at[pl.ds(c * CHUNK, CHUNK)]],
                rows[b], sem_g[b], add=True).wait()
            issue_store(c, b)

        for c in range(LEAD):
            issue_load(c, c % NBUF)
        for c in range(LEAD):
            process(c, c % NBUF)
            issue_load(c + LEAD, (c + LEAD) % NBUF)

        def body(g, carry):
            for j in range(NBUF):
                c = LEAD + g * NBUF + j
                b = (LEAD + j) % NBUF
                b2 = (LEAD + j + LEAD) % NBUF
                process(c, b)
                wait_store(c - LEAD, b2)
                issue_load(c + LEAD, b2)
            return carry

        lax.fori_loop(0, n_steady // NBUF, body, 0)

        for k in range(LEAD):
            c = n_chunks - LEAD + k
            process(c, c % NBUF)
            wait_store(c - LEAD, (c + LEAD) % NBUF)
        for k in range(LEAD):
            c = n_chunks - LEAD + k
            wait_store(c, c % NBUF)

    return sc_call


def kernel(x, rel_times):
    b, t, d = x.shape
    n_rows = b * t
    x2 = x.reshape(n_rows, d)
    idx = rel_times.reshape(n_rows).astype(jnp.int32)
    pe = jnp.asarray(_PE_TABLE)
    out = _make_sc_call(n_rows)(x2, pe, idx)
    return out.reshape(b, t, d)


# speedup vs baseline: 7.1838x; 1.0308x over previous
import functools

import jax
import jax.numpy as jnp
import numpy as np
from jax import lax
from jax.experimental import pallas as pl
from jax.experimental.pallas import tpu as pltpu
from jax.experimental.pallas import tpu_sc as plsc

D_MODEL = 128
MAX_LEN = 5000

NUM_CORES = 2
NUM_SUBCORES = 16
NUM_WORKERS = NUM_CORES * NUM_SUBCORES

CHUNK = 128
NBUF = 5
LOAD_AHEAD = 3
DRAIN_LAG = 2


def _build_pe_np():
    position = np.arange(MAX_LEN, dtype=np.float32)[:, None]
    div_term = np.exp(
        np.arange(0, D_MODEL, 2, dtype=np.float32) * (-np.log(10000.0) / D_MODEL)
    ).astype(np.float32)
    pe_input = (position * div_term).astype(np.float32)
    pe = np.zeros((MAX_LEN, D_MODEL), dtype=np.float32)
    pe[:, ::2] = np.sin(pe_input)
    pe[:, 1::2] = np.cos(pe_input)
    return pe


_PE_TABLE = _build_pe_np()


def _make_sc_call(n_rows: int):
    rows_per_worker = n_rows // NUM_WORKERS
    n_chunks = rows_per_worker // CHUNK
    n_steady = n_chunks - LOAD_AHEAD - DRAIN_LAG
    assert rows_per_worker * NUM_WORKERS == n_rows
    assert n_chunks * CHUNK == rows_per_worker
    assert n_steady % NBUF == 0 and n_steady >= NBUF
    assert LOAD_AHEAD + DRAIN_LAG == NBUF

    mesh = plsc.VectorSubcoreMesh(
        core_axis_name="c", subcore_axis_name="s",
        num_cores=NUM_CORES, num_subcores=NUM_SUBCORES,
    )

    @functools.partial(
        pl.kernel,
        out_type=jax.ShapeDtypeStruct((n_rows, D_MODEL), jnp.float32),
        mesh=mesh,
        scratch_types=[
            pltpu.VMEM((rows_per_worker,), jnp.int32),
            [pltpu.VMEM((CHUNK, D_MODEL), jnp.float32) for _ in range(NBUF)],
            [pltpu.SemaphoreType.DMA for _ in range(NBUF)],
            [pltpu.SemaphoreType.DMA for _ in range(NBUF)],
            [pltpu.SemaphoreType.DMA for _ in range(NBUF)],
        ],
    )
    def sc_call(x_hbm, pe_hbm, idx_hbm, out_hbm, idx_all, rows, sem_x, sem_g, sem_w):
        wid = lax.axis_index("s") * NUM_CORES + lax.axis_index("c")
        w_base = wid * rows_per_worker

        pltpu.sync_copy(idx_hbm.at[pl.ds(w_base, rows_per_worker)], idx_all)

        def issue_load(c, b):
            pltpu.async_copy(
                x_hbm.at[pl.ds(w_base + c * CHUNK, CHUNK)], rows[b], sem_x[b])

        def wait_load(c, b):
            pltpu.make_async_copy(
                x_hbm.at[pl.ds(w_base + c * CHUNK, CHUNK)], rows[b],
                sem_x[b]).wait()

        def issue_store(c, b):
            pltpu.async_copy(
                rows[b], out_hbm.at[pl.ds(w_base + c * CHUNK, CHUNK)], sem_w[b])

        def wait_store(c, b):
            pltpu.make_async_copy(
                rows[b], out_hbm.at[pl.ds(w_base + c * CHUNK, CHUNK)],
                sem_w[b]).wait()

        def issue_gather(c, b):
            pltpu.async_copy(
                pe_hbm.at[idx_all.at[pl.ds(c * CHUNK, CHUNK)]],
                rows[b], sem_g[b], add=True)

        def wait_gather(c, b):
            pltpu.make_async_copy(
                pe_hbm.at[idx_all.at[pl.ds(c * CHUNK, CHUNK)]],
                rows[b], sem_g[b]).wait()

        def visit(c, r, has_next_gather, has_drain, has_load):
            if has_next_gather:
                wait_load(c + 1, (r + 1) % NBUF)
                issue_gather(c + 1, (r + 1) % NBUF)
            wait_gather(c, r)
            issue_store(c, r)
            if has_drain:
                wait_store(c - DRAIN_LAG, (r - DRAIN_LAG) % NBUF)
            if has_load:
                issue_load(c + LOAD_AHEAD, (r + LOAD_AHEAD) % NBUF)

        for c in range(LOAD_AHEAD):
            issue_load(c, c % NBUF)
        wait_load(0, 0)
        issue_gather(0, 0)
        for c in range(DRAIN_LAG):
            visit(c, c % NBUF, True, False, True)

        def body(g, carry):
            for j in range(NBUF):
                c = DRAIN_LAG + g * NBUF + j
                visit(c, (DRAIN_LAG + j) % NBUF, True, True, True)
            return carry

        lax.fori_loop(0, n_steady // NBUF, body, 0)

        for c in range(n_chunks - LOAD_AHEAD, n_chunks):
            visit(c, c % NBUF, c + 1 < n_chunks, True, False)
        for c in range(n_chunks - DRAIN_LAG, n_chunks):
            wait_store(c, c % NBUF)

    return sc_call


def kernel(x, rel_times):
    b, t, d = x.shape
    n_rows = b * t
    x2 = x.reshape(n_rows, d)
    idx = rel_times.reshape(n_rows).astype(jnp.int32)
    pe = jnp.asarray(_PE_TABLE)
    out = _make_sc_call(n_rows)(x2, pe, idx)
    return out.reshape(b, t, d)


# speedup vs baseline: 10.3444x; 1.4400x over previous
import functools

import jax
import jax.numpy as jnp
import numpy as np
from jax import lax
from jax.experimental import pallas as pl
from jax.experimental.pallas import tpu as pltpu
from jax.experimental.pallas import tpu_sc as plsc

D_MODEL = 128
MAX_LEN = 5000

NUM_CORES = 2
NUM_SUBCORES = 16
NUM_WORKERS = NUM_CORES * NUM_SUBCORES

CHUNK = 128
NBUF = 5
LOAD_AHEAD = 3
DRAIN_LAG = 2


def _build_pe_np():
    position = np.arange(MAX_LEN, dtype=np.float32)[:, None]
    div_term = np.exp(
        np.arange(0, D_MODEL, 2, dtype=np.float32) * (-np.log(10000.0) / D_MODEL)
    ).astype(np.float32)
    pe_input = (position * div_term).astype(np.float32)
    pe = np.zeros((MAX_LEN, D_MODEL), dtype=np.float32)
    pe[:, ::2] = np.sin(pe_input)
    pe[:, 1::2] = np.cos(pe_input)
    return pe


_PE_TABLE = _build_pe_np()


def _make_sc_call(n_rows: int):
    rows_per_worker = n_rows // NUM_WORKERS
    n_chunks = rows_per_worker // CHUNK
    n_steady = n_chunks - LOAD_AHEAD - DRAIN_LAG
    assert rows_per_worker * NUM_WORKERS == n_rows
    assert n_chunks * CHUNK == rows_per_worker
    assert n_steady % NBUF == 0 and n_steady >= NBUF
    assert LOAD_AHEAD + DRAIN_LAG == NBUF

    mesh = plsc.VectorSubcoreMesh(
        core_axis_name="c", subcore_axis_name="s",
        num_cores=NUM_CORES, num_subcores=NUM_SUBCORES,
    )

    @functools.partial(
        pl.kernel,
        out_type=jax.ShapeDtypeStruct((n_rows, D_MODEL), jnp.float32),
        mesh=mesh,
        scratch_types=[
            [pltpu.VMEM((CHUNK,), jnp.int32) for _ in range(NBUF)],
            [pltpu.VMEM((CHUNK, D_MODEL), jnp.float32) for _ in range(NBUF)],
            pltpu.VMEM_SHARED((MAX_LEN, D_MODEL), jnp.float32),
            [pltpu.SemaphoreType.DMA for _ in range(NBUF)],
            [pltpu.SemaphoreType.DMA for _ in range(NBUF)],
            [pltpu.SemaphoreType.DMA for _ in range(NBUF)],
            [pltpu.SemaphoreType.DMA for _ in range(NBUF)],
        ],
    )
    def sc_call(x_hbm, pe_hbm, idx_hbm, out_hbm, idx_v, rows, pe_sh,
                sem_x, sem_i, sem_g, sem_w):
        sid = lax.axis_index("s")
        wid = sid * NUM_CORES + lax.axis_index("c")
        w_base = wid * rows_per_worker

        @pl.when(sid == 0)
        def _():
            pltpu.sync_copy(pe_hbm, pe_sh)

        plsc.subcore_barrier()

        def issue_load(c, b):
            pltpu.async_copy(
                idx_hbm.at[pl.ds(w_base + c * CHUNK, CHUNK)], idx_v[b],
                sem_i[b])
            pltpu.async_copy(
                x_hbm.at[pl.ds(w_base + c * CHUNK, CHUNK)], rows[b], sem_x[b])

        def wait_load(c, b):
            pltpu.make_async_copy(
                idx_hbm.at[pl.ds(w_base + c * CHUNK, CHUNK)], idx_v[b],
                sem_i[b]).wait()
            pltpu.make_async_copy(
                x_hbm.at[pl.ds(w_base + c * CHUNK, CHUNK)], rows[b],
                sem_x[b]).wait()

        def issue_store(c, b):
            pltpu.async_copy(
                rows[b], out_hbm.at[pl.ds(w_base + c * CHUNK, CHUNK)], sem_w[b])

        def wait_store(c, b):
            pltpu.make_async_copy(
                rows[b], out_hbm.at[pl.ds(w_base + c * CHUNK, CHUNK)],
                sem_w[b]).wait()

        def issue_gather(c, b):
            pltpu.async_copy(pe_sh.at[idx_v[b]], rows[b], sem_g[b], add=True)

        def wait_gather(c, b):
            pltpu.make_async_copy(
                pe_sh.at[idx_v[b]], rows[b], sem_g[b]).wait()

        def visit(c, r, has_next_gather, has_drain, has_load):
            if has_next_gather:
                wait_load(c + 1, (r + 1) % NBUF)
                issue_gather(c + 1, (r + 1) % NBUF)
            wait_gather(c, r)
            issue_store(c, r)
            if has_drain:
                wait_store(c - DRAIN_LAG, (r - DRAIN_LAG) % NBUF)
            if has_load:
                issue_load(c + LOAD_AHEAD, (r + LOAD_AHEAD) % NBUF)

        for c in range(LOAD_AHEAD):
            issue_load(c, c % NBUF)
        wait_load(0, 0)
        issue_gather(0, 0)
        for c in range(DRAIN_LAG):
            visit(c, c % NBUF, True, False, True)

        def body(g, carry):
            for j in range(NBUF):
                c = DRAIN_LAG + g * NBUF + j
                visit(c, (DRAIN_LAG + j) % NBUF, True, True, True)
            return carry

        lax.fori_loop(0, n_steady // NBUF, body, 0)

        for c in range(n_chunks - LOAD_AHEAD, n_chunks):
            visit(c, c % NBUF, c + 1 < n_chunks, True, False)
        for c in range(n_chunks - DRAIN_LAG, n_chunks):
            wait_store(c, c % NBUF)

    return sc_call


def kernel(x, rel_times):
    b, t, d = x.shape
    n_rows = b * t
    x2 = x.reshape(n_rows, d)
    idx = rel_times.reshape(n_rows).astype(jnp.int32)
    pe = jnp.asarray(_PE_TABLE)
    out = _make_sc_call(n_rows)(x2, pe, idx)
    return out.reshape(b, t, d)


# speedup vs baseline: 10.4291x; 1.0082x over previous
import functools

import jax
import jax.numpy as jnp
import numpy as np
from jax import lax
from jax.experimental import pallas as pl
from jax.experimental.pallas import tpu as pltpu
from jax.experimental.pallas import tpu_sc as plsc

D_MODEL = 128
MAX_LEN = 5000

NUM_CORES = 2
NUM_SUBCORES = 16
NUM_WORKERS = NUM_CORES * NUM_SUBCORES

CHUNK = 128
NBUF = 5
LOAD_AHEAD = 3
DRAIN_LAG = 2


def _build_pe_np():
    position = np.arange(MAX_LEN, dtype=np.float32)[:, None]
    div_term = np.exp(
        np.arange(0, D_MODEL, 2, dtype=np.float32) * (-np.log(10000.0) / D_MODEL)
    ).astype(np.float32)
    pe_input = (position * div_term).astype(np.float32)
    pe = np.zeros((MAX_LEN, D_MODEL), dtype=np.float32)
    pe[:, ::2] = np.sin(pe_input)
    pe[:, 1::2] = np.cos(pe_input)
    return pe


_PE_TABLE = _build_pe_np()


def _make_sc_call(n_rows: int):
    rows_per_worker = n_rows // NUM_WORKERS
    n_chunks = rows_per_worker // CHUNK
    n_steady = n_chunks - LOAD_AHEAD - DRAIN_LAG
    assert rows_per_worker * NUM_WORKERS == n_rows
    assert n_chunks * CHUNK == rows_per_worker
    assert n_steady % NBUF == 0 and n_steady >= NBUF
    assert LOAD_AHEAD + DRAIN_LAG == NBUF

    mesh = plsc.VectorSubcoreMesh(
        core_axis_name="c", subcore_axis_name="s",
        num_cores=NUM_CORES, num_subcores=NUM_SUBCORES,
    )

    @functools.partial(
        pl.kernel,
        out_type=jax.ShapeDtypeStruct((n_rows, D_MODEL), jnp.float32),
        mesh=mesh,
        scratch_types=[
            [pltpu.VMEM((CHUNK,), jnp.int32) for _ in range(NBUF)],
            [pltpu.VMEM((CHUNK, D_MODEL), jnp.float32) for _ in range(NBUF)],
            pltpu.VMEM_SHARED((MAX_LEN, D_MODEL), jnp.float32),
            [pltpu.SemaphoreType.DMA for _ in range(NBUF)],
            [pltpu.SemaphoreType.DMA for _ in range(NBUF)],
            [pltpu.SemaphoreType.DMA for _ in range(NBUF)],
            [pltpu.SemaphoreType.DMA for _ in range(NBUF)],
            [pltpu.SemaphoreType.DMA for _ in range(NBUF)],
        ],
    )
    def sc_call(x_hbm, pe_hbm, idx_hbm, out_hbm, idx_v, rows, pe_sh,
                sem_x, sem_i, sem_g, sem_g2, sem_w):
        sid = lax.axis_index("s")
        wid = sid * NUM_CORES + lax.axis_index("c")
        w_base = wid * rows_per_worker

        @pl.when(sid == 0)
        def _():
            pltpu.sync_copy(pe_hbm, pe_sh)

        plsc.subcore_barrier()

        def issue_load(c, b):
            pltpu.async_copy(
                idx_hbm.at[pl.ds(w_base + c * CHUNK, CHUNK)], idx_v[b],
                sem_i[b])
            pltpu.async_copy(
                x_hbm.at[pl.ds(w_base + c * CHUNK, CHUNK)], rows[b], sem_x[b])

        def wait_load(c, b):
            pltpu.make_async_copy(
                idx_hbm.at[pl.ds(w_base + c * CHUNK, CHUNK)], idx_v[b],
                sem_i[b]).wait()
            pltpu.make_async_copy(
                x_hbm.at[pl.ds(w_base + c * CHUNK, CHUNK)], rows[b],
                sem_x[b]).wait()

        def issue_store(c, b):
            pltpu.async_copy(
                rows[b], out_hbm.at[pl.ds(w_base + c * CHUNK, CHUNK)], sem_w[b])

        def wait_store(c, b):
            pltpu.make_async_copy(
                rows[b], out_hbm.at[pl.ds(w_base + c * CHUNK, CHUNK)],
                sem_w[b]).wait()

        HALF = CHUNK // 2

        def issue_gather(c, b):
            pltpu.async_copy(
                pe_sh.at[idx_v[b].at[pl.ds(0, HALF)]],
                rows[b].at[pl.ds(0, HALF)], sem_g[b], add=True)
            pltpu.async_copy(
                pe_sh.at[idx_v[b].at[pl.ds(HALF, HALF)]],
                rows[b].at[pl.ds(HALF, HALF)], sem_g2[b], add=True)

        def wait_gather(c, b):
            pltpu.make_async_copy(
                pe_sh.at[idx_v[b].at[pl.ds(0, HALF)]],
                rows[b].at[pl.ds(0, HALF)], sem_g[b]).wait()
            pltpu.make_async_copy(
                pe_sh.at[idx_v[b].at[pl.ds(HALF, HALF)]],
                rows[b].at[pl.ds(HALF, HALF)], sem_g2[b]).wait()

        def visit(c, r, has_next_gather, has_drain, has_load):
            if has_next_gather:
                wait_load(c + 1, (r + 1) % NBUF)
                issue_gather(c + 1, (r + 1) % NBUF)
            wait_gather(c, r)
            issue_store(c, r)
            if has_drain:
                wait_store(c - DRAIN_LAG, (r - DRAIN_LAG) % NBUF)
            if has_load:
                issue_load(c + LOAD_AHEAD, (r + LOAD_AHEAD) % NBUF)

        for c in range(LOAD_AHEAD):
            issue_load(c, c % NBUF)
        wait_load(0, 0)
        issue_gather(0, 0)
        for c in range(DRAIN_LAG):
            visit(c, c % NBUF, True, False, True)

        def body(g, carry):
            for j in range(NBUF):
                c = DRAIN_LAG + g * NBUF + j
                visit(c, (DRAIN_LAG + j) % NBUF, True, True, True)
            return carry

        lax.fori_loop(0, n_steady // NBUF, body, 0)

        for c in range(n_chunks - LOAD_AHEAD, n_chunks):
            visit(c, c % NBUF, c + 1 < n_chunks, True, False)
        for c in range(n_chunks - DRAIN_LAG, n_chunks):
            wait_store(c, c % NBUF)

    return sc_call


def kernel(x, rel_times):
    b, t, d = x.shape
    n_rows = b * t
    x2 = x.reshape(n_rows, d)
    idx = rel_times.reshape(n_rows).astype(jnp.int32)
    pe = jnp.asarray(_PE_TABLE)
    out = _make_sc_call(n_rows)(x2, pe, idx)
    return out.reshape(b, t, d)
